# Initial kernel scaffold; baseline (speedup 1.0000x reference)
import jax
import jax.numpy as jnp
from jax.experimental import pallas as pl


def kernel(x, emb_w, W1, b1, Wh, bh, Wo, bo):
    raise NotImplementedError("write your pallas kernel here")


# speedup vs baseline: 2.0621x; 2.0621x over previous
import jax
import jax.numpy as jnp
from jax import lax
from jax.experimental import pallas as pl
from jax.experimental.pallas import tpu as pltpu

_B = 1024
_D = 64
_N = _B * _D
_K = 1024
_E = 64
_T = 1024
_G = _N // _T

_BF = jnp.bfloat16
_F = jnp.float32

_CONTRACT_1_1 = (((1,), (1,)), ((), ()))


def _vq_body(flat_ref, xt_ref, emb_ref, ind_ref, loss_ref, acc):
    g = pl.program_id(0)
    f = flat_ref[0]
    xt = xt_ref[0]
    embw = emb_ref[...]

    exps = lax.broadcasted_iota(jnp.int32, (1, _E), 1).astype(_F) + 1.0
    xres = jnp.exp2(exps * jnp.log2(f))

    sm = jnp.sum(xres, axis=1, keepdims=True)
    sq = jnp.sum(xres * xres, axis=1, keepdims=True)

    ones = jnp.ones((1, _E), _F)
    embn = lax.dot_general(ones, embw * embw, _CONTRACT_1_1,
                           precision=lax.Precision.HIGHEST)

    mm = lax.dot_general(xres.astype(_BF), embw.astype(_BF), _CONTRACT_1_1,
                         preferred_element_type=_F)
    dist = (sm + embn) - 2.0 * mm

    minv = jnp.min(dist, axis=1, keepdims=True)
    lane = lax.broadcasted_iota(jnp.int32, (_T, _K), 1).astype(_F)
    indf = jnp.min(jnp.where(dist == minv, lane, float(_K)),
                   axis=1, keepdims=True)
    ind_ref[0] = indf

    qpart = jnp.sum(minv - sm + sq)
    epart = jnp.sum((xt - indf) ** 2)

    @pl.when(g == 0)
    def _():
        acc[0] = qpart
        acc[1] = epart

    @pl.when(g > 0)
    def _():
        acc[0] += qpart
        acc[1] += epart

    @pl.when(g == _G - 1)
    def _():
        loss_ref[0, 0] = acc[0] / (_N * _E) + 0.25 * (acc[1] / _N)


def _mlp_body(q_ref, x_ref, w1_ref, b1_ref, wh_ref, bh_ref, wo_ref, bo_ref,
              f_ref):
    qv = q_ref[...]
    xv = x_ref[...]
    qs = xv + (qv - xv)

    h = jnp.maximum(
        lax.dot_general(qs.astype(_BF), w1_ref[...].astype(_BF),
                        _CONTRACT_1_1, preferred_element_type=_F)
        + b1_ref[...], 0.0)
    whb = wh_ref[...].astype(_BF)
    bh = bh_ref[...]
    for _ in range(4):
        h = jnp.maximum(
            lax.dot_general(h.astype(_BF), whb, _CONTRACT_1_1,
                            preferred_element_type=_F) + bh, 0.0)
    f_ref[...] = jnp.maximum(
        lax.dot_general(h.astype(_BF), wo_ref[...].astype(_BF),
                        _CONTRACT_1_1, preferred_element_type=_F)
        + bo_ref[...], 0.0)


def kernel(x, emb_w, W1, b1, Wh, bh, Wo, bo):
    flat3d = x.reshape(_G, _T, 1)
    xt3d = x.T.reshape(_G, _T, 1)

    ind3d, loss = pl.pallas_call(
        _vq_body,
        grid=(_G,),
        in_specs=[
            pl.BlockSpec((1, _T, 1), lambda i: (i, 0, 0)),
            pl.BlockSpec((1, _T, 1), lambda i: (i, 0, 0)),
            pl.BlockSpec((_K, _E), lambda i: (0, 0)),
        ],
        out_specs=[
            pl.BlockSpec((1, _T, 1), lambda i: (i, 0, 0)),
            pl.BlockSpec((1, 1), lambda i: (0, 0), memory_space=pltpu.SMEM),
        ],
        out_shape=[
            jax.ShapeDtypeStruct((_G, _T, 1), _F),
            jax.ShapeDtypeStruct((1, 1), _F),
        ],
        scratch_shapes=[pltpu.SMEM((2,), _F)],
        compiler_params=pltpu.CompilerParams(
            dimension_semantics=("arbitrary",)),
    )(flat3d, xt3d, emb_w)

    q = ind3d.reshape(_D, _B).T

    f = pl.pallas_call(
        _mlp_body,
        out_shape=jax.ShapeDtypeStruct((_B, _D), _F),
    )(q, x, W1, b1.reshape(1, -1), Wh, bh.reshape(1, -1),
      Wo, bo.reshape(1, -1))

    return f, loss.reshape(())


# speedup vs baseline: 2.2218x; 1.0775x over previous
import jax
import jax.numpy as jnp
from jax import lax
from jax.experimental import pallas as pl
from jax.experimental.pallas import tpu as pltpu

_B = 1024
_D = 64
_N = _B * _D
_K = 1024
_E = 64
_T = 1024
_G = _N // _T

_BF = jnp.bfloat16
_F = jnp.float32

_CONTRACT_1_1 = (((1,), (1,)), ((), ()))


def _vq_body(flat_ref, xt_ref, emb_ref, ind_ref, loss_ref, acc, embn_v):
    g = pl.program_id(0)
    f = flat_ref[0]
    xt = xt_ref[0]
    embw = emb_ref[...]

    @pl.when(g == 0)
    def _():
        ones = jnp.ones((1, _E), _F)
        embn_v[...] = lax.dot_general(ones, embw * embw, _CONTRACT_1_1,
                                      precision=lax.Precision.HIGHEST)

    exps = lax.broadcasted_iota(jnp.int32, (1, _E), 1).astype(_F) + 1.0
    xres = jnp.exp2(exps * jnp.log2(f))

    sm = jnp.sum(xres, axis=1, keepdims=True)
    sq = jnp.sum(xres * xres, axis=1, keepdims=True)

    mm2 = lax.dot_general((-2.0 * xres).astype(_BF), embw.astype(_BF),
                          _CONTRACT_1_1, preferred_element_type=_F)
    d2 = embn_v[...] + mm2

    minv = jnp.min(d2, axis=1, keepdims=True)
    lane = lax.broadcasted_iota(jnp.int32, (_T, _K), 1)
    indi = jnp.min(jnp.where(d2 == minv, lane, _K),
                   axis=1, keepdims=True)
    indf = indi.astype(_F)
    ind_ref[0] = indf

    qpart = jnp.sum(minv + sq)
    epart = jnp.sum((xt - indf) ** 2)

    @pl.when(g == 0)
    def _():
        acc[0] = qpart
        acc[1] = epart

    @pl.when(g > 0)
    def _():
        acc[0] += qpart
        acc[1] += epart

    @pl.when(g == _G - 1)
    def _():
        loss_ref[0, 0] = acc[0] / (_N * _E) + 0.25 * (acc[1] / _N)


def _mlp_body(q_ref, x_ref, w1_ref, b1_ref, wh_ref, bh_ref, wo_ref, bo_ref,
              f_ref):
    qv = q_ref[...]
    xv = x_ref[...]
    qs = xv + (qv - xv)

    h = jnp.maximum(
        lax.dot_general(qs.astype(_BF), w1_ref[...].astype(_BF),
                        _CONTRACT_1_1, preferred_element_type=_F)
        + b1_ref[...], 0.0)
    whb = wh_ref[...].astype(_BF)
    bh = bh_ref[...]
    for _ in range(4):
        h = jnp.maximum(
            lax.dot_general(h.astype(_BF), whb, _CONTRACT_1_1,
                            preferred_element_type=_F) + bh, 0.0)
    f_ref[...] = jnp.maximum(
        lax.dot_general(h.astype(_BF), wo_ref[...].astype(_BF),
                        _CONTRACT_1_1, preferred_element_type=_F)
        + bo_ref[...], 0.0)


def kernel(x, emb_w, W1, b1, Wh, bh, Wo, bo):
    flat3d = x.reshape(_G, _T, 1)
    xt3d = x.T.reshape(_G, _T, 1)

    ind3d, loss = pl.pallas_call(
        _vq_body,
        grid=(_G,),
        in_specs=[
            pl.BlockSpec((1, _T, 1), lambda i: (i, 0, 0)),
            pl.BlockSpec((1, _T, 1), lambda i: (i, 0, 0)),
            pl.BlockSpec((_K, _E), lambda i: (0, 0)),
        ],
        out_specs=[
            pl.BlockSpec((1, _T, 1), lambda i: (i, 0, 0)),
            pl.BlockSpec((1, 1), lambda i: (0, 0), memory_space=pltpu.SMEM),
        ],
        out_shape=[
            jax.ShapeDtypeStruct((_G, _T, 1), _F),
            jax.ShapeDtypeStruct((1, 1), _F),
        ],
        scratch_shapes=[pltpu.SMEM((2,), _F), pltpu.VMEM((1, _K), _F)],
        compiler_params=pltpu.CompilerParams(
            dimension_semantics=("arbitrary",)),
    )(flat3d, xt3d, emb_w)

    q = ind3d.reshape(_D, _B).T

    f = pl.pallas_call(
        _mlp_body,
        out_shape=jax.ShapeDtypeStruct((_B, _D), _F),
    )(q, x, W1, b1.reshape(1, -1), Wh, bh.reshape(1, -1),
      Wo, bo.reshape(1, -1))

    return f, loss.reshape(())


# speedup vs baseline: 3.7857x; 1.7039x over previous
import jax
import jax.numpy as jnp
from jax import lax
from jax.experimental import pallas as pl
from jax.experimental.pallas import tpu as pltpu

_B = 1024
_D = 64
_N = _B * _D
_K = 1024
_E = 64
_T = 1024
_G = _N // _T

_BF = jnp.bfloat16
_F = jnp.float32

_CONTRACT_1_1 = (((1,), (1,)), ((), ()))


def _vq_body(flat_ref, xt_ref, emb_ref, ind_ref, loss_ref, acc, embn_v):
    g = pl.program_id(0)
    f = flat_ref[0]
    xt = xt_ref[0]
    embw = emb_ref[...]

    @pl.when(g == 0)
    def _():
        embn_v[...] = jnp.sum(embw * embw, axis=1, keepdims=True)

    exps = lax.broadcasted_iota(jnp.int32, (_E, 1), 0).astype(_F) + 1.0
    xres = jnp.exp2(exps * jnp.log2(f))

    sm = jnp.sum(xres, axis=0, keepdims=True)
    sq = jnp.sum(xres * xres, axis=0, keepdims=True)

    mm2 = lax.dot_general(embw.astype(_BF), (-2.0 * xres).astype(_BF),
                          (((1,), (0,)), ((), ())),
                          preferred_element_type=_F)
    d2 = embn_v[...] + mm2

    minv = jnp.min(d2, axis=0, keepdims=True)
    lane = lax.broadcasted_iota(jnp.int32, (_K, _T), 0)
    indi = jnp.min(jnp.where(d2 == minv, lane, _K),
                   axis=0, keepdims=True)
    indf = indi.astype(_F)
    ind_ref[0] = indf

    qpart = jnp.sum(minv + sq)
    epart = jnp.sum((xt - indf) ** 2)

    @pl.when(g == 0)
    def _():
        acc[0] = qpart
        acc[1] = epart

    @pl.when(g > 0)
    def _():
        acc[0] += qpart
        acc[1] += epart

    @pl.when(g == _G - 1)
    def _():
        loss_ref[0, 0] = acc[0] / (_N * _E) + 0.25 * (acc[1] / _N)


def _mlp_body(q_ref, x_ref, w1_ref, b1_ref, wh_ref, bh_ref, wo_ref, bo_ref,
              f_ref):
    qv = q_ref[...]
    xv = x_ref[...]
    qs = xv + (qv - xv)

    h = jnp.maximum(
        lax.dot_general(qs.astype(_BF), w1_ref[...].astype(_BF),
                        _CONTRACT_1_1, preferred_element_type=_F)
        + b1_ref[...], 0.0)
    whb = wh_ref[...].astype(_BF)
    bh = bh_ref[...]
    for _ in range(4):
        h = jnp.maximum(
            lax.dot_general(h.astype(_BF), whb, _CONTRACT_1_1,
                            preferred_element_type=_F) + bh, 0.0)
    f_ref[...] = jnp.maximum(
        lax.dot_general(h.astype(_BF), wo_ref[...].astype(_BF),
                        _CONTRACT_1_1, preferred_element_type=_F)
        + bo_ref[...], 0.0)


def kernel(x, emb_w, W1, b1, Wh, bh, Wo, bo):
    flat3d = x.reshape(_G, 1, _T)
    xt3d = x.T.reshape(_G, 1, _T)

    ind3d, loss = pl.pallas_call(
        _vq_body,
        grid=(_G,),
        in_specs=[
            pl.BlockSpec((1, 1, _T), lambda i: (i, 0, 0)),
            pl.BlockSpec((1, 1, _T), lambda i: (i, 0, 0)),
            pl.BlockSpec((_K, _E), lambda i: (0, 0)),
        ],
        out_specs=[
            pl.BlockSpec((1, 1, _T), lambda i: (i, 0, 0)),
            pl.BlockSpec((1, 1), lambda i: (0, 0), memory_space=pltpu.SMEM),
        ],
        out_shape=[
            jax.ShapeDtypeStruct((_G, 1, _T), _F),
            jax.ShapeDtypeStruct((1, 1), _F),
        ],
        scratch_shapes=[pltpu.SMEM((2,), _F), pltpu.VMEM((_K, 1), _F)],
        compiler_params=pltpu.CompilerParams(
            dimension_semantics=("arbitrary",)),
    )(flat3d, xt3d, emb_w)

    q = ind3d.reshape(_D, _B).T

    f = pl.pallas_call(
        _mlp_body,
        out_shape=jax.ShapeDtypeStruct((_B, _D), _F),
    )(q, x, W1, b1.reshape(1, -1), Wh, bh.reshape(1, -1),
      Wo, bo.reshape(1, -1))

    return f, loss.reshape(())


# speedup vs baseline: 3.8518x; 1.0175x over previous
import jax
import jax.numpy as jnp
from jax import lax
from jax.experimental import pallas as pl
from jax.experimental.pallas import tpu as pltpu

_B = 1024
_D = 64
_N = _B * _D
_K = 1024
_E = 64
_T = 1024
_G = _N // _T

_BF = jnp.bfloat16
_F = jnp.float32

_CONTRACT_1_1 = (((1,), (1,)), ((), ()))


def _embn_body(emb_ref, embn_ref):
    embw = emb_ref[...]
    embn_ref[...] = jnp.sum(embw * embw, axis=1, keepdims=True)


def _vq_body(flat_ref, xt_ref, emb_ref, embn_ref, kiota_ref,
             ind_ref, loss_ref, acc):
    g = pl.program_id(0)
    f = flat_ref[0]
    xt = xt_ref[0]
    embw = emb_ref[...]
    embn = embn_ref[...]

    exps = lax.broadcasted_iota(jnp.int32, (_E, 1), 0).astype(_F) + 1.0
    xres = jnp.exp2(exps * jnp.log2(f))

    sm = jnp.sum(xres, axis=0, keepdims=True)
    sq = jnp.sum(xres * xres, axis=0, keepdims=True)

    mm2 = lax.dot_general(embw.astype(_BF), (-2.0 * xres).astype(_BF),
                          (((1,), (0,)), ((), ())),
                          preferred_element_type=_F)

    minv = jnp.min(embn + mm2, axis=0, keepdims=True)
    indf = jnp.min(jnp.where((mm2 + embn) == minv, kiota_ref[...], float(_K)),
                   axis=0, keepdims=True)
    ind_ref[0] = indf

    qpart = jnp.sum(minv + sq)
    epart = jnp.sum((xt - indf) ** 2)

    @pl.when(g == 0)
    def _():
        acc[0] = qpart
        acc[1] = epart

    @pl.when(g > 0)
    def _():
        acc[0] += qpart
        acc[1] += epart

    @pl.when(g == _G - 1)
    def _():
        loss_ref[0, 0] = acc[0] / (_N * _E) + 0.25 * (acc[1] / _N)


def _mlp_body(q_ref, x_ref, w1_ref, b1_ref, wh_ref, bh_ref, wo_ref, bo_ref,
              f_ref):
    qv = q_ref[...]
    xv = x_ref[...]
    qs = xv + (qv - xv)

    h = jnp.maximum(
        lax.dot_general(qs.astype(_BF), w1_ref[...].astype(_BF),
                        _CONTRACT_1_1, preferred_element_type=_F)
        + b1_ref[...], 0.0)
    whb = wh_ref[...].astype(_BF)
    bh = bh_ref[...]
    for _ in range(4):
        h = jnp.maximum(
            lax.dot_general(h.astype(_BF), whb, _CONTRACT_1_1,
                            preferred_element_type=_F) + bh, 0.0)
    f_ref[...] = jnp.maximum(
        lax.dot_general(h.astype(_BF), wo_ref[...].astype(_BF),
                        _CONTRACT_1_1, preferred_element_type=_F)
        + bo_ref[...], 0.0)


def kernel(x, emb_w, W1, b1, Wh, bh, Wo, bo):
    flat3d = x.reshape(_G, 1, _T)
    xt3d = x.T.reshape(_G, 1, _T)
    kiota = lax.broadcasted_iota(_F, (_K, _T), 0)

    embn = pl.pallas_call(
        _embn_body,
        out_shape=jax.ShapeDtypeStruct((_K, 1), _F),
    )(emb_w)

    ind3d, loss = pl.pallas_call(
        _vq_body,
        grid=(_G,),
        in_specs=[
            pl.BlockSpec((1, 1, _T), lambda i: (i, 0, 0)),
            pl.BlockSpec((1, 1, _T), lambda i: (i, 0, 0)),
            pl.BlockSpec((_K, _E), lambda i: (0, 0)),
            pl.BlockSpec((_K, 1), lambda i: (0, 0)),
            pl.BlockSpec((_K, _T), lambda i: (0, 0)),
        ],
        out_specs=[
            pl.BlockSpec((1, 1, _T), lambda i: (i, 0, 0)),
            pl.BlockSpec((1, 1), lambda i: (0, 0), memory_space=pltpu.SMEM),
        ],
        out_shape=[
            jax.ShapeDtypeStruct((_G, 1, _T), _F),
            jax.ShapeDtypeStruct((1, 1), _F),
        ],
        scratch_shapes=[pltpu.SMEM((2,), _F)],
        compiler_params=pltpu.CompilerParams(
            dimension_semantics=("arbitrary",)),
    )(flat3d, xt3d, emb_w, embn, kiota)

    q = ind3d.reshape(_D, _B).T

    f = pl.pallas_call(
        _mlp_body,
        out_shape=jax.ShapeDtypeStruct((_B, _D), _F),
    )(q, x, W1, b1.reshape(1, -1), Wh, bh.reshape(1, -1),
      Wo, bo.reshape(1, -1))

    return f, loss.reshape(())


# speedup vs baseline: 4.2776x; 1.1105x over previous
import jax
import jax.numpy as jnp
from jax import lax
from jax.experimental import pallas as pl
from jax.experimental.pallas import tpu as pltpu

_B = 1024
_D = 64
_N = _B * _D
_K = 1024
_E = 64
_T = 1024
_G = _N // _T

_BF = jnp.bfloat16
_F = jnp.float32

_CONTRACT_1_1 = (((1,), (1,)), ((), ()))


def _embn_body(emb_ref, embn_ref):
    embw = emb_ref[...]
    embn_ref[...] = jnp.sum(embw * embw, axis=1, keepdims=True)


def _vq_body(flat_ref, xt_ref, emb_ref, embn_ref, kiota_ref,
             ind_ref, loss_ref, acc):
    g = pl.program_id(0)
    f = flat_ref[0]
    xt = xt_ref[0]
    embw = emb_ref[...]
    embn = embn_ref[...]

    exps = lax.broadcasted_iota(jnp.int32, (_E, 1), 0).astype(_F) + 1.0
    xres = jnp.exp2(exps * jnp.log2(f))

    sm = jnp.sum(xres, axis=0, keepdims=True)
    sq = jnp.sum(xres * xres, axis=0, keepdims=True)

    mm2 = lax.dot_general(embw.astype(_BF), (-2.0 * xres).astype(_BF),
                          (((1,), (0,)), ((), ())),
                          preferred_element_type=_F)

    kiota = kiota_ref[...]
    h = _K // 2
    keep = (embn[:h] + mm2[:h]) <= (embn[h:] + mm2[h:])
    v = jnp.minimum(embn[:h] + mm2[:h], embn[h:] + mm2[h:])
    ii = jnp.where(keep, kiota[:h], kiota[h:])
    rows = h
    while rows > 8:
        h = rows // 2
        keep = v[:h] <= v[h:]
        ii = jnp.where(keep, ii[:h], ii[h:])
        v = jnp.minimum(v[:h], v[h:])
        rows = h
    minv = jnp.min(v, axis=0, keepdims=True)
    indf = jnp.min(jnp.where(v == minv, ii, float(_K)),
                   axis=0, keepdims=True)
    ind_ref[0] = indf

    qpart = jnp.sum(minv + sq)
    epart = jnp.sum((xt - indf) ** 2)

    @pl.when(g == 0)
    def _():
        acc[0] = qpart
        acc[1] = epart

    @pl.when(g > 0)
    def _():
        acc[0] += qpart
        acc[1] += epart

    @pl.when(g == _G - 1)
    def _():
        loss_ref[0, 0] = acc[0] / (_N * _E) + 0.25 * (acc[1] / _N)


def _mlp_body(q_ref, x_ref, w1_ref, b1_ref, wh_ref, bh_ref, wo_ref, bo_ref,
              f_ref):
    qv = q_ref[...]
    xv = x_ref[...]
    qs = xv + (qv - xv)

    h = jnp.maximum(
        lax.dot_general(qs.astype(_BF), w1_ref[...].astype(_BF),
                        _CONTRACT_1_1, preferred_element_type=_F)
        + b1_ref[...], 0.0)
    whb = wh_ref[...].astype(_BF)
    bh = bh_ref[...]
    for _ in range(4):
        h = jnp.maximum(
            lax.dot_general(h.astype(_BF), whb, _CONTRACT_1_1,
                            preferred_element_type=_F) + bh, 0.0)
    f_ref[...] = jnp.maximum(
        lax.dot_general(h.astype(_BF), wo_ref[...].astype(_BF),
                        _CONTRACT_1_1, preferred_element_type=_F)
        + bo_ref[...], 0.0)


def kernel(x, emb_w, W1, b1, Wh, bh, Wo, bo):
    flat3d = x.reshape(_G, 1, _T)
    xt3d = x.T.reshape(_G, 1, _T)
    kiota = lax.broadcasted_iota(_F, (_K, _T), 0)

    embn = pl.pallas_call(
        _embn_body,
        out_shape=jax.ShapeDtypeStruct((_K, 1), _F),
    )(emb_w)

    ind3d, loss = pl.pallas_call(
        _vq_body,
        grid=(_G,),
        in_specs=[
            pl.BlockSpec((1, 1, _T), lambda i: (i, 0, 0)),
            pl.BlockSpec((1, 1, _T), lambda i: (i, 0, 0)),
            pl.BlockSpec((_K, _E), lambda i: (0, 0)),
            pl.BlockSpec((_K, 1), lambda i: (0, 0)),
            pl.BlockSpec((_K, _T), lambda i: (0, 0)),
        ],
        out_specs=[
            pl.BlockSpec((1, 1, _T), lambda i: (i, 0, 0)),
            pl.BlockSpec((1, 1), lambda i: (0, 0), memory_space=pltpu.SMEM),
        ],
        out_shape=[
            jax.ShapeDtypeStruct((_G, 1, _T), _F),
            jax.ShapeDtypeStruct((1, 1), _F),
        ],
        scratch_shapes=[pltpu.SMEM((2,), _F)],
        compiler_params=pltpu.CompilerParams(
            dimension_semantics=("arbitrary",)),
    )(flat3d, xt3d, emb_w, embn, kiota)

    q = ind3d.reshape(_D, _B).T

    f = pl.pallas_call(
        _mlp_body,
        out_shape=jax.ShapeDtypeStruct((_B, _D), _F),
    )(q, x, W1, b1.reshape(1, -1), Wh, bh.reshape(1, -1),
      Wo, bo.reshape(1, -1))

    return f, loss.reshape(())


# speedup vs baseline: 4.5057x; 1.0533x over previous
import jax
import jax.numpy as jnp
from jax import lax
from jax.experimental import pallas as pl
from jax.experimental.pallas import tpu as pltpu

_B = 1024
_D = 64
_N = _B * _D
_K = 1024
_E = 64
_T = 1024
_G = _N // _T

_BF = jnp.bfloat16
_F = jnp.float32

_CONTRACT_1_1 = (((1,), (1,)), ((), ()))


def _embn_body(emb_ref, embn_ref):
    embw = emb_ref[...]
    embn_ref[...] = jnp.sum(embw * embw, axis=1, keepdims=True)


def _vq_body(flat_ref, emb_ref, embn_ref, ind_ref, qsum_ref, acc):
    g = pl.program_id(0)
    f = flat_ref[0]
    embw = emb_ref[...]
    embn = embn_ref[...]

    exps = lax.broadcasted_iota(jnp.int32, (_E, 1), 0).astype(_F) + 1.0
    xres = jnp.exp2(exps * jnp.log2(f))

    sm = jnp.sum(xres, axis=0, keepdims=True)
    sq = jnp.sum(xres * xres, axis=0, keepdims=True)

    mm2 = lax.dot_general(embw.astype(_BF), (-2.0 * xres).astype(_BF),
                          (((1,), (0,)), ((), ())),
                          preferred_element_type=_F)

    h = _K // 2
    av = embn[:h] + mm2[:h]
    bv = embn[h:] + mm2[h:]
    keep = av <= bv
    v = jnp.minimum(av, bv)
    kio = lax.broadcasted_iota(jnp.int32, (h, _T), 0)
    ii = jnp.where(keep, kio, kio + h)
    rows = h
    while rows > 8:
        h = rows // 2
        keep = v[:h] <= v[h:]
        ii = jnp.where(keep, ii[:h], ii[h:])
        v = jnp.minimum(v[:h], v[h:])
        rows = h
    minv = jnp.min(v, axis=0, keepdims=True)
    indf = jnp.min(jnp.where(v == minv, ii.astype(_F), float(_K)),
                   axis=0, keepdims=True)
    ind_ref[0] = indf

    qpart = jnp.sum(minv + sq)

    @pl.when(g == 0)
    def _():
        acc[0] = qpart

    @pl.when(g > 0)
    def _():
        acc[0] += qpart

    @pl.when(g == _G - 1)
    def _():
        qsum_ref[0, 0] = acc[0]


def _mlp_body(i2d_ref, x_ref, qsum_ref, w1_ref, b1_ref, wh_ref, bh_ref,
              wo_ref, bo_ref, f_ref, loss_ref):
    qv = jnp.transpose(i2d_ref[...], (1, 0))
    xv = x_ref[...]

    esum = jnp.sum((xv - qv) ** 2)
    loss_ref[0, 0] = qsum_ref[0, 0] / (_N * _E) + 0.25 * (esum / _N)

    qs = xv + (qv - xv)
    h = jnp.maximum(
        lax.dot_general(qs.astype(_BF), w1_ref[...].astype(_BF),
                        _CONTRACT_1_1, preferred_element_type=_F)
        + b1_ref[...], 0.0)
    whb = wh_ref[...].astype(_BF)
    bh = bh_ref[...]
    for _ in range(4):
        h = jnp.maximum(
            lax.dot_general(h.astype(_BF), whb, _CONTRACT_1_1,
                            preferred_element_type=_F) + bh, 0.0)
    f_ref[...] = jnp.maximum(
        lax.dot_general(h.astype(_BF), wo_ref[...].astype(_BF),
                        _CONTRACT_1_1, preferred_element_type=_F)
        + bo_ref[...], 0.0)


def kernel(x, emb_w, W1, b1, Wh, bh, Wo, bo):
    flat3d = x.reshape(_G, 1, _T)

    embn = pl.pallas_call(
        _embn_body,
        out_shape=jax.ShapeDtypeStruct((_K, 1), _F),
    )(emb_w)

    ind3d, qsum = pl.pallas_call(
        _vq_body,
        grid=(_G,),
        in_specs=[
            pl.BlockSpec((1, 1, _T), lambda i: (i, 0, 0)),
            pl.BlockSpec((_K, _E), lambda i: (0, 0)),
            pl.BlockSpec((_K, 1), lambda i: (0, 0)),
        ],
        out_specs=[
            pl.BlockSpec((1, 1, _T), lambda i: (i, 0, 0)),
            pl.BlockSpec((1, 1), lambda i: (0, 0), memory_space=pltpu.SMEM),
        ],
        out_shape=[
            jax.ShapeDtypeStruct((_G, 1, _T), _F),
            jax.ShapeDtypeStruct((1, 1), _F),
        ],
        scratch_shapes=[pltpu.SMEM((1,), _F)],
        compiler_params=pltpu.CompilerParams(
            dimension_semantics=("arbitrary",)),
    )(flat3d, emb_w, embn)

    f, loss = pl.pallas_call(
        _mlp_body,
        in_specs=[
            pl.BlockSpec((_D, _B), lambda: (0, 0)),
            pl.BlockSpec((_B, _D), lambda: (0, 0)),
            pl.BlockSpec((1, 1), lambda: (0, 0), memory_space=pltpu.SMEM),
            pl.BlockSpec((_B, _D), lambda: (0, 0)),
            pl.BlockSpec((1, _B), lambda: (0, 0)),
            pl.BlockSpec((_B, _B), lambda: (0, 0)),
            pl.BlockSpec((1, _B), lambda: (0, 0)),
            pl.BlockSpec((_D, _B), lambda: (0, 0)),
            pl.BlockSpec((1, _D), lambda: (0, 0)),
        ],
        out_specs=[
            pl.BlockSpec((_B, _D), lambda: (0, 0)),
            pl.BlockSpec((1, 1), lambda: (0, 0), memory_space=pltpu.SMEM),
        ],
        out_shape=[
            jax.ShapeDtypeStruct((_B, _D), _F),
            jax.ShapeDtypeStruct((1, 1), _F),
        ],
    )(ind3d.reshape(_D, _B), x, qsum, W1, b1.reshape(1, -1), Wh,
      bh.reshape(1, -1), Wo, bo.reshape(1, -1))

    return f, loss.reshape(())
